# baseline (speedup 1.0000x reference)
import functools

import jax
import jax.numpy as jnp
from jax import lax
from jax.experimental import pallas as pl
from jax.experimental.pallas import tpu as pltpu
from jax.experimental.pallas import tpu_sc as plsc

H = 128
W = 128
N = H * W
NW = 32
CHUNK = N // NW
L = 16
VECS = CHUNK // L


def _ifloor(x):
    t = x.astype(jnp.int32)
    return t - (t.astype(jnp.float32) > x).astype(jnp.int32)


_mesh = plsc.VectorSubcoreMesh(core_axis_name="c", subcore_axis_name="s")


@functools.partial(
    pl.kernel,
    mesh=_mesh,
    compiler_params=pltpu.CompilerParams(needs_layout_passes=False),
    out_type=jax.ShapeDtypeStruct((N,), jnp.float32),
    scratch_types=[
        pltpu.VMEM((N,), jnp.float32),
        pltpu.VMEM((CHUNK,), jnp.float32),
        pltpu.VMEM((CHUNK,), jnp.float32),
        pltpu.VMEM((CHUNK,), jnp.float32),
    ],
)
def _warp(img_hbm, disp_hbm, out_hbm, img_v, dh_v, dw_v, out_v):
    wid = lax.axis_index("s") * 2 + lax.axis_index("c")
    base = wid * CHUNK
    pltpu.sync_copy(img_hbm, img_v)
    pltpu.sync_copy(disp_hbm.at[pl.ds(base, CHUNK)], dh_v)
    pltpu.sync_copy(disp_hbm.at[pl.ds(N + base, CHUNK)], dw_v)

    lane = lax.broadcasted_iota(jnp.int32, (L,), 0)
    row0f = jnp.broadcast_to(wid * (CHUNK // W), (L,)).astype(jnp.float32)

    for it in range(VECS):
        off = it * L
        hu = dh_v[pl.ds(off, L)] + row0f + jnp.float32(off // W + 1)
        wu = (dw_v[pl.ds(off, L)]
              + (lane + (off % W + 1)).astype(jnp.float32))
        hf_u = _ifloor(hu)
        wf_u = _ifloor(wu)
        ihf = jnp.clip(hf_u - 1, 0, H - 1) * W
        ihc = jnp.clip(hf_u, 0, H - 1) * W
        iwf = jnp.clip(wf_u - 1, 0, W - 1)
        iwc = jnp.clip(wf_u, 0, W - 1)
        vhf = (hf_u >= 1) & (hf_u <= H)
        vhc = (hf_u >= 0) & (hf_u <= H - 1)
        vwf = (wf_u >= 1) & (wf_u <= W)
        vwc = (wf_u >= 0) & (wf_u <= W - 1)
        d_h = jnp.clip(hf_u + 1, 0, H + 1).astype(jnp.float32) - hu
        d_w = jnp.clip(wf_u + 1, 0, W + 1).astype(jnp.float32) - wu
        zero = jnp.float32(0.0)
        one = jnp.float32(1.0)

        def corner(ih, iw, valid):
            v = plsc.load_gather(img_v, [ih + iw])
            return jnp.where(valid, v, zero)

        v00 = corner(ihf, iwf, vhf & vwf)
        v10 = corner(ihc, iwf, vhc & vwf)
        v01 = corner(ihf, iwc, vhf & vwc)
        v11 = corner(ihc, iwc, vhc & vwc)
        out = (v00 * (d_w * d_h) + v10 * (d_w * (one - d_h))
               + v01 * ((one - d_w) * d_h) + v11 * ((one - d_w) * (one - d_h)))
        out_v[pl.ds(off, L)] = out

    pltpu.sync_copy(out_v, out_hbm.at[pl.ds(base, CHUNK)])


def kernel(input1, input2):
    img = input1.reshape(N)
    disp = input2.reshape(2 * N)
    out = _warp(img, disp)
    return out.reshape(1, 1, H, W)


# speedup vs baseline: 1.0529x; 1.0529x over previous
import functools

import jax
import jax.numpy as jnp
from jax import lax
from jax.experimental import pallas as pl
from jax.experimental.pallas import tpu as pltpu
from jax.experimental.pallas import tpu_sc as plsc

H = 128
W = 128
N = H * W
NW = 32
CHUNK = N // NW
L = 16
VECS = CHUNK // L


def _ifloor(x):
    t = x.astype(jnp.int32)
    return t - (t.astype(jnp.float32) > x).astype(jnp.int32)


_mesh = plsc.VectorSubcoreMesh(core_axis_name="c", subcore_axis_name="s")


@functools.partial(
    pl.kernel,
    mesh=_mesh,
    compiler_params=pltpu.CompilerParams(needs_layout_passes=False),
    out_type=jax.ShapeDtypeStruct((N,), jnp.float32),
    scratch_types=[
        pltpu.VMEM((N,), jnp.float32),
        pltpu.VMEM((CHUNK,), jnp.float32),
        pltpu.VMEM((CHUNK,), jnp.float32),
        pltpu.VMEM((CHUNK,), jnp.float32),
    ],
)
def _warp(img_hbm, disp_hbm, out_hbm, img_v, dh_v, dw_v, out_v):
    wid = lax.axis_index("s") * 2 + lax.axis_index("c")
    base = wid * CHUNK
    pltpu.sync_copy(img_hbm, img_v)
    pltpu.sync_copy(disp_hbm.at[pl.ds(base, CHUNK)], dh_v)
    pltpu.sync_copy(disp_hbm.at[pl.ds(N + base, CHUNK)], dw_v)

    lane = lax.broadcasted_iota(jnp.int32, (L,), 0)
    row0f = jnp.broadcast_to(wid * (CHUNK // W), (L,)).astype(jnp.float32)

    @plsc.parallel_loop(0, CHUNK, step=L, unroll=4)
    def body(off):
        row_off = jnp.broadcast_to(off // W + 1, (L,))
        hu = dh_v[pl.ds(off, L)] + row0f + row_off.astype(jnp.float32)
        wu = dw_v[pl.ds(off, L)] + (lane + (off % W + 1)).astype(jnp.float32)
        hf_u = _ifloor(hu)
        wf_u = _ifloor(wu)
        ihf = jnp.clip(hf_u - 1, 0, H - 1) * W
        ihc = jnp.clip(hf_u, 0, H - 1) * W
        iwf = jnp.clip(wf_u - 1, 0, W - 1)
        iwc = jnp.clip(wf_u, 0, W - 1)
        vhf = (hf_u >= 1) & (hf_u <= H)
        vhc = (hf_u >= 0) & (hf_u <= H - 1)
        vwf = (wf_u >= 1) & (wf_u <= W)
        vwc = (wf_u >= 0) & (wf_u <= W - 1)
        d_h = jnp.clip(hf_u + 1, 0, H + 1).astype(jnp.float32) - hu
        d_w = jnp.clip(wf_u + 1, 0, W + 1).astype(jnp.float32) - wu
        zero = jnp.float32(0.0)
        one = jnp.float32(1.0)

        def corner(ih, iw, valid):
            v = plsc.load_gather(img_v, [ih + iw])
            return jnp.where(valid, v, zero)

        v00 = corner(ihf, iwf, vhf & vwf)
        v10 = corner(ihc, iwf, vhc & vwf)
        v01 = corner(ihf, iwc, vhf & vwc)
        v11 = corner(ihc, iwc, vhc & vwc)
        out = (v00 * (d_w * d_h) + v10 * (d_w * (one - d_h))
               + v01 * ((one - d_w) * d_h) + v11 * ((one - d_w) * (one - d_h)))
        out_v[pl.ds(off, L)] = out

    pltpu.sync_copy(out_v, out_hbm.at[pl.ds(base, CHUNK)])


def kernel(input1, input2):
    img = input1.reshape(N)
    disp = input2.reshape(2 * N)
    out = _warp(img, disp)
    return out.reshape(1, 1, H, W)
